# baseline (device time: 28101 ns/iter reference)
import jax
import jax.numpy as jnp
from jax import lax
from jax.experimental import pallas as pl
from jax.experimental.pallas import tpu as pltpu

N_DEV = 16
B, SQ, D = 2, 128, 512
HQ_LOCAL, DH = 8, 64
ROWS = B * SQ
HALF = ROWS // 2


def kernel(x, Wq, Wo, K_ext, V_ext):
    d_model = Wo.shape[1]

    ROUNDS = [
        ("rs", 2, 64, 128),
        ("bf", 4, 64, 192),
        ("bf", 8, 64, 256),
        ("ag", 2, 64, 320),
        ("ag", 1, 128, 384),
    ]

    def body(x_ref, wq_ref, wo_ref, k_ref, v_ref, out_ref,
             recv_ref, send_buf_ref, send_sems, recv_sems):
        my = lax.axis_index("i")
        bit0 = my & 1

        barrier_sem = pltpu.get_barrier_semaphore()
        for xr in (1, 2, 4, 8):
            pl.semaphore_signal(barrier_sem, inc=1,
                                device_id=(my ^ xr,),
                                device_id_type=pl.DeviceIdType.MESH)

        def compute_half(b):
            xb = x_ref[b]
            qb = jnp.dot(xb, wq_ref[...],
                         preferred_element_type=jnp.float32)
            q3 = qb.reshape(SQ, HQ_LOCAL, DH)
            kv = k_ref[b]
            vv = v_ref[b]
            outs_h = []
            for h in range(HQ_LOCAL):
                qh = q3[:, h, :]
                kh = kv[:, h, :]
                sc = jnp.dot(qh, kh.T,
                             preferred_element_type=jnp.float32) * 0.125
                sc = sc - jnp.max(sc, axis=-1, keepdims=True)
                p = jnp.exp(sc)
                p = p / jnp.sum(p, axis=-1, keepdims=True)
                outs_h.append(jnp.dot(p, vv[:, h, :],
                                      preferred_element_type=jnp.float32))
            attn_b = jnp.concatenate(outs_h, axis=-1)
            out_ref[pl.ds(b * HALF, HALF)] = jnp.dot(
                attn_b, wo_ref[...], preferred_element_type=jnp.float32)

        @pl.when(bit0 == 0)
        def _():
            compute_half(1)

        @pl.when(bit0 == 1)
        def _():
            compute_half(0)

        pl.semaphore_wait(barrier_sem, 4)

        send_start0 = (1 - bit0) * HALF
        send_buf_ref[pl.ds(0, HALF)] = out_ref[
            pl.ds(send_start0, HALF)].astype(jnp.bfloat16)
        rdma0 = pltpu.make_async_remote_copy(
            src_ref=send_buf_ref.at[pl.ds(0, HALF)],
            dst_ref=recv_ref.at[pl.ds(0, HALF)],
            send_sem=send_sems.at[0],
            recv_sem=recv_sems.at[0],
            device_id=(my ^ 1,),
            device_id_type=pl.DeviceIdType.MESH,
        )
        rdma0.start()

        @pl.when(bit0 == 0)
        def _():
            compute_half(0)

        @pl.when(bit0 == 1)
        def _():
            compute_half(1)

        rdma0.wait()
        s = bit0 * HALF
        out_ref[pl.ds(s, HALF)] = (
            out_ref[pl.ds(s, HALF)]
            + recv_ref[pl.ds(0, HALF)].astype(jnp.float32)
        )

        for idx, (kind, xr, L, off) in enumerate(ROUNDS, start=1):
            partner = my ^ xr
            bit = (my & xr) // xr
            if kind == "rs":
                src_start = s + (1 - bit) * L
            else:
                src_start = s
            send_buf_ref[pl.ds(0, L)] = out_ref[
                pl.ds(src_start, L)].astype(jnp.bfloat16)
            rdma = pltpu.make_async_remote_copy(
                src_ref=send_buf_ref.at[pl.ds(0, L)],
                dst_ref=recv_ref.at[pl.ds(off, L)],
                send_sem=send_sems.at[idx],
                recv_sem=recv_sems.at[idx],
                device_id=(partner,),
                device_id_type=pl.DeviceIdType.MESH,
            )
            rdma.start()
            rdma.wait()
            if kind == "rs":
                s = s + bit * L
            if kind in ("rs", "bf"):
                out_ref[pl.ds(s, L)] = (
                    out_ref[pl.ds(s, L)]
                    + recv_ref[pl.ds(off, L)].astype(jnp.float32)
                )
            else:
                ps = s + L - 2 * bit * L
                out_ref[pl.ds(ps, L)] = recv_ref[
                    pl.ds(off, L)].astype(jnp.float32)
                s = s - bit * L

    out = pl.pallas_call(
        body,
        out_shape=jax.ShapeDtypeStruct((ROWS, d_model), jnp.float32),
        in_specs=[pl.BlockSpec(memory_space=pltpu.VMEM)] * 5,
        out_specs=pl.BlockSpec(memory_space=pltpu.VMEM),
        scratch_shapes=[
            pltpu.VMEM((512, d_model), jnp.bfloat16),
            pltpu.VMEM((HALF, d_model), jnp.bfloat16),
            pltpu.SemaphoreType.DMA((6,)),
            pltpu.SemaphoreType.DMA((6,)),
        ],
        compiler_params=pltpu.CompilerParams(collective_id=0),
    )(x, Wq, Wo, K_ext, V_ext)
    return out.reshape(B, SQ, d_model)
